# baseline (speedup 1.0000x reference)
import jax
import jax.numpy as jnp
from jax.experimental import pallas as pl
from jax.experimental.pallas import tpu as pltpu

B = 256
J = 17
S = 96 * 72
LN = J * S
TOPK = 8
CH = 8
NCH = B // CH
NBUF = 4


def _stream_body(x_hbm, y_hbm, o_ref, xbuf, ybuf, xsem, ysem):
    def start(i, slot):
        pltpu.make_async_copy(
            x_hbm.at[pl.ds(i * CH, CH), :], xbuf.at[slot], xsem.at[slot]
        ).start()
        pltpu.make_async_copy(
            y_hbm.at[pl.ds(i * CH, CH), :], ybuf.at[slot], ysem.at[slot]
        ).start()

    for s in range(NBUF):
        start(s, s)

    def step(i, carry):
        slot = jax.lax.rem(i, NBUF)
        pltpu.make_async_copy(
            x_hbm.at[pl.ds(i * CH, CH), :], xbuf.at[slot], xsem.at[slot]
        ).wait()
        pltpu.make_async_copy(
            y_hbm.at[pl.ds(i * CH, CH), :], ybuf.at[slot], ysem.at[slot]
        ).wait()
        d = xbuf[slot] - ybuf[slot]
        d2 = d * d
        for j in range(J):
            s = jnp.sum(d2[:, j * S:(j + 1) * S], axis=1, keepdims=True)
            o_ref[pl.ds(i * CH, CH), j:j + 1] = s * (0.5 / S)

        @pl.when(i + NBUF < NCH)
        def _():
            start(i + NBUF, slot)

        return carry

    jax.lax.fori_loop(0, NCH, step, 0)


def _mine_body(l_ref, o_ref):
    l = l_ref[...]
    jidx = jax.lax.broadcasted_iota(jnp.int32, (J, B), 0)
    rank = jnp.zeros((J, B), jnp.int32)
    for k in range(J):
        lk = l[k:k + 1, :]
        gt = (lk > l) | ((lk == l) & (k < jidx))
        rank = rank + gt.astype(jnp.int32)
    topsum = jnp.sum(jnp.where(rank < TOPK, l, 0.0))
    o_ref[...] = topsum[None, None] * (1.0 / (TOPK * B))


def kernel(output, target):
    x = output.reshape(B, LN)
    y = target.reshape(B, LN)
    losses = pl.pallas_call(
        _stream_body,
        in_specs=[
            pl.BlockSpec(memory_space=pltpu.HBM),
            pl.BlockSpec(memory_space=pltpu.HBM),
        ],
        out_specs=pl.BlockSpec((B, J), lambda: (0, 0)),
        out_shape=jax.ShapeDtypeStruct((B, J), jnp.float32),
        scratch_shapes=[
            pltpu.VMEM((NBUF, CH, LN), jnp.float32),
            pltpu.VMEM((NBUF, CH, LN), jnp.float32),
            pltpu.SemaphoreType.DMA((NBUF,)),
            pltpu.SemaphoreType.DMA((NBUF,)),
        ],
    )(x, y)
    out = pl.pallas_call(
        _mine_body,
        out_shape=jax.ShapeDtypeStruct((1, 1), jnp.float32),
    )(losses.T)
    return out[0, 0]


# speedup vs baseline: 1.0964x; 1.0964x over previous
import jax
import jax.numpy as jnp
from jax.experimental import pallas as pl
from jax.experimental.pallas import tpu as pltpu

B = 256
J = 17
S = 96 * 72
LN = J * S
TOPK = 8
BB = 8


def _sums_body(d_ref, o_ref):
    d = d_ref[...]
    d2 = d * d
    for j in range(J):
        s = jnp.sum(d2[:, j * S:(j + 1) * S], axis=1, keepdims=True)
        o_ref[:, j:j + 1] = s * (0.5 / S)


def _mine_body(l_ref, o_ref):
    l = l_ref[...]
    jidx = jax.lax.broadcasted_iota(jnp.int32, (J, B), 0)
    rank = jnp.zeros((J, B), jnp.int32)
    for k in range(J):
        lk = l[k:k + 1, :]
        gt = (lk > l) | ((lk == l) & (k < jidx))
        rank = rank + gt.astype(jnp.int32)
    topsum = jnp.sum(jnp.where(rank < TOPK, l, 0.0))
    o_ref[...] = topsum[None, None] * (1.0 / (TOPK * B))


def kernel(output, target):
    x = output.reshape(B, LN)
    y = target.reshape(B, LN)
    d = x - y
    losses = pl.pallas_call(
        _sums_body,
        grid=(B // BB,),
        in_specs=[pl.BlockSpec((BB, LN), lambda i: (i, 0))],
        out_specs=pl.BlockSpec((BB, J), lambda i: (i, 0)),
        out_shape=jax.ShapeDtypeStruct((B, J), jnp.float32),
        compiler_params=pltpu.CompilerParams(
            allow_input_fusion=[True],
        ),
    )(d)
    out = pl.pallas_call(
        _mine_body,
        out_shape=jax.ShapeDtypeStruct((1, 1), jnp.float32),
    )(losses.T)
    return out[0, 0]
